# baseline (speedup 1.0000x reference)
import jax
import jax.numpy as jnp
from jax import lax
from jax.experimental import pallas as pl

B, D, K = 4096, 64, 8192
DT = 2 * D
BB = 256


def _quantise_body(mu_ref, ls_ref, t_ref, dists_ref, dist_ref, qmu_ref, qsig_ref):
    mu = mu_ref[...]
    sig = jnp.exp(ls_ref[...])
    rr = lax.broadcasted_iota(jnp.int32, (D, DT), 0)
    cc = lax.broadcasted_iota(jnp.int32, (D, DT), 1)
    se_t = (cc == 2 * rr).astype(jnp.float32)
    so_t = (cc == 2 * rr + 1).astype(jnp.float32)
    xdn = (((1,), (0,)), ((), ()))
    x = (lax.dot_general(mu, se_t, xdn, preferred_element_type=jnp.float32)
         + lax.dot_general(sig, so_t, xdn, preferred_element_type=jnp.float32))
    t = t_ref[...]

    dn = (((1,), (1,)), ((), ()))
    cross = lax.dot_general(x, t, dn,
                            preferred_element_type=jnp.float32)
    n1 = jnp.sum(x * x, axis=1, keepdims=True)
    n2 = jnp.sum(t * t, axis=1)
    d = n1 + n2[None, :] - 2.0 * cross
    dists_ref[...] = d

    row_min = jnp.min(d, axis=1, keepdims=True)
    dist_ref[...] = row_min
    col = lax.broadcasted_iota(jnp.int32, d.shape, 1)
    row_arg = jnp.min(jnp.where(d == row_min, col, K), axis=1)
    onehot = (col == row_arg[:, None]).astype(jnp.float32)
    qdn = (((1,), (0,)), ((), ()))
    qp = lax.dot_general(onehot, t, qdn,
                         preferred_element_type=jnp.float32)
    r = lax.broadcasted_iota(jnp.int32, (DT, D), 0)
    c = lax.broadcasted_iota(jnp.int32, (DT, D), 1)
    s_even = (r == 2 * c).astype(jnp.float32)
    s_odd = (r == 2 * c + 1).astype(jnp.float32)
    qmu_ref[...] = lax.dot_general(qp, s_even, qdn,
                                   preferred_element_type=jnp.float32)
    qsig_ref[...] = lax.dot_general(qp, s_odd, qdn,
                                    preferred_element_type=jnp.float32)


def _quantise(input_mu, input_logsig, table):
    return pl.pallas_call(
        _quantise_body,
        grid=(B // BB,),
        in_specs=[
            pl.BlockSpec((BB, D), lambda i: (i, 0)),
            pl.BlockSpec((BB, D), lambda i: (i, 0)),
            pl.BlockSpec((K, DT), lambda i: (0, 0)),
        ],
        out_specs=[
            pl.BlockSpec((BB, K), lambda i: (i, 0)),
            pl.BlockSpec((BB, 1), lambda i: (i, 0)),
            pl.BlockSpec((BB, D), lambda i: (i, 0)),
            pl.BlockSpec((BB, D), lambda i: (i, 0)),
        ],
        out_shape=[
            jax.ShapeDtypeStruct((B, K), jnp.float32),
            jax.ShapeDtypeStruct((B, 1), jnp.float32),
            jax.ShapeDtypeStruct((B, D), jnp.float32),
            jax.ShapeDtypeStruct((B, D), jnp.float32),
        ],
    )(input_mu, input_logsig, table)


@jax.jit
def kernel(input_mu, input_logsig, on_states):
    table = on_states.reshape(K, DT)
    dists, dist, qmu, qsig = _quantise(input_mu, input_logsig, table)
    return ((qmu, qsig), dists, dist)
